# baseline (speedup 1.0000x reference)
import functools

import jax
import jax.numpy as jnp
from jax import lax
from jax.experimental import pallas as pl
from jax.experimental.pallas import tpu as pltpu
from jax.experimental.pallas import tpu_sc as plsc

REP = 8


def kernel(x, pe_weight):
    batch = x.shape[0]
    max_len, dim = pe_weight.shape
    flat = max_len * dim
    info = plsc.get_sparse_core_info()
    nw = info.num_cores * info.num_subcores
    b_per_w = batch // nw
    steps = b_per_w // REP
    pe2d = pe_weight.reshape(1, flat)

    mesh = plsc.VectorSubcoreMesh(core_axis_name="c", subcore_axis_name="s")

    @functools.partial(
        pl.kernel,
        mesh=mesh,
        out_type=jax.ShapeDtypeStruct((batch, flat), pe_weight.dtype),
        scratch_types=[
            pltpu.VMEM((REP, flat), pe_weight.dtype),
            pltpu.SemaphoreType.DMA,
        ],
    )
    def sc_fill(pe_hbm, out_hbm, buf, sem):
        wid = lax.axis_index("s") * info.num_cores + lax.axis_index("c")
        base = wid * b_per_w
        loads = [
            pltpu.async_copy(pe_hbm, buf.at[pl.ds(r, 1)], sem)
            for r in range(REP)
        ]
        for ld in loads:
            ld.wait()
        stores = [
            pltpu.async_copy(
                buf, out_hbm.at[pl.ds(base + s * REP, REP)], sem
            )
            for s in range(steps)
        ]
        for st in stores:
            st.wait()

    out2d = sc_fill(pe2d)
    return out2d.reshape(batch, max_len, dim)


# speedup vs baseline: 4.0705x; 4.0705x over previous
import jax
import jax.numpy as jnp
from jax.experimental import pallas as pl

L_BLOCK = 8


def _splat_kernel(pe_ref, out_ref):
    out_ref[...] = jnp.broadcast_to(pe_ref[...], out_ref.shape)


def kernel(x, pe_weight):
    batch = x.shape[0]
    max_len, dim = pe_weight.shape
    pe3d = pe_weight.reshape(max_len, dim, 1)
    out_t = pl.pallas_call(
        _splat_kernel,
        grid=(max_len // L_BLOCK,),
        in_specs=[pl.BlockSpec((L_BLOCK, dim, 1), lambda i: (i, 0, 0))],
        out_specs=pl.BlockSpec((L_BLOCK, dim, batch), lambda i: (i, 0, 0)),
        out_shape=jax.ShapeDtypeStruct((max_len, dim, batch), pe_weight.dtype),
    )(pe3d)
    return out_t.transpose(2, 0, 1)
